# baseline (speedup 1.0000x reference)
import functools

import jax
import jax.numpy as jnp
from jax import lax
from jax.experimental import pallas as pl
from jax.experimental.pallas import tpu as pltpu
from jax.experimental.pallas import tpu_sc as plsc

B, CTX, D, V = 1024, 20, 64, 100000

NC, NS, L = 2, 16, 16
NW = NC * NS
BPW = B // NW
RPW = BPW * CTX
DP = 128

V_TILE = 4096
NV = (V + V_TILE - 1) // V_TILE
LAST_VALID = V - (NV - 1) * V_TILE


VP = NV * V_TILE


def _tc_prep_table(embT, b2):
    def kern(e_ref, b_ref, o_ref, eb_ref):
        o_ref[...] = jnp.concatenate(
            [
                jnp.transpose(e_ref[...], (1, 0)),
                jnp.zeros((V_TILE, DP - D), jnp.float32),
            ],
            axis=1,
        )
        i = pl.program_id(0)
        col = lax.broadcasted_iota(jnp.int32, (1, V_TILE), 1)
        eb_ref[...] = jnp.where(
            col + i * V_TILE < V, jnp.exp(b_ref[...]), 0.0
        )

    return pl.pallas_call(
        kern,
        grid=(NV,),
        in_specs=[
            pl.BlockSpec((D, V_TILE), lambda i: (0, i)),
            pl.BlockSpec((1, V_TILE), lambda i: (0, i)),
        ],
        out_specs=[
            pl.BlockSpec((V_TILE, DP), lambda i: (i, 0)),
            pl.BlockSpec((1, V_TILE), lambda i: (0, i)),
        ],
        out_shape=[
            jax.ShapeDtypeStruct((V, DP), jnp.float32),
            jax.ShapeDtypeStruct((1, VP), jnp.float32),
        ],
    )(embT, b2)


def _sc_embed_sum(xT, embp):
    mesh = plsc.VectorSubcoreMesh(core_axis_name="c", subcore_axis_name="s")

    @functools.partial(
        pl.kernel,
        mesh=mesh,
        out_type=jax.ShapeDtypeStruct((B, D), jnp.float32),
        scratch_types=[
            pltpu.VMEM((CTX, B), jnp.int32),
            pltpu.VMEM((RPW, DP), jnp.float32),
            pltpu.VMEM((BPW, D), jnp.float32),
            pltpu.SemaphoreType.DMA,
        ],
    )
    def k(x_hbm, emb_hbm, out_hbm, idx_v, rows_v, acc_v, sem):
        wid = lax.axis_index("s") * NC + lax.axis_index("c")
        base = wid * BPW
        pltpu.sync_copy(x_hbm, idx_v)
        copies = [
            pltpu.async_copy(
                emb_hbm.at[idx_v.at[j, pl.ds(base, BPW)]],
                rows_v.at[pl.ds(j * BPW, BPW)],
                sem,
            )
            for j in range(CTX)
        ]
        for cp in copies:
            cp.wait()

        def body(bi, carry):
            for g in range(D // L):
                acc = rows_v[bi, pl.ds(g * L, L)]
                for j in range(1, CTX):
                    acc = acc + rows_v[j * BPW + bi, pl.ds(g * L, L)]
                acc_v[bi, pl.ds(g * L, L)] = acc
            return carry

        lax.fori_loop(0, BPW, body, 0)
        pltpu.sync_copy(acc_v, out_hbm.at[pl.ds(base, BPW)])

    return k(xT, embp)


def _tc_stats(s, xT, emb0, WT, ebp):
    def kern(s_ref, x_ref, e0_ref, w_ref, b_ref, m_ref, lse_ref, lrun):
        v = pl.program_id(0)

        @pl.when(v == 0)
        def _():
            n0row = jnp.sum(
                jnp.where(x_ref[...] == 0, 1.0, 0.0), axis=0, keepdims=True
            )
            n0 = jnp.transpose(n0row, (1, 0))
            m_ref[...] = (s_ref[...] - n0 * e0_ref[...]) * (1.0 / CTX)
            lrun[...] = jnp.zeros((1, B), jnp.float32)

        ex = jnp.exp(
            lax.dot_general(
                w_ref[...],
                m_ref[...],
                (((0,), (1,)), ((), ())),
                preferred_element_type=jnp.float32,
            )
        )
        lrun[...] = lrun[...] + lax.dot_general(
            b_ref[...],
            ex,
            (((1,), (0,)), ((), ())),
            preferred_element_type=jnp.float32,
        )

        @pl.when(v == NV - 1)
        def _():
            lse_ref[...] = jnp.log(lrun[...])

    return pl.pallas_call(
        kern,
        grid=(NV,),
        in_specs=[
            pl.BlockSpec((B, D), lambda v: (0, 0)),
            pl.BlockSpec((CTX, B), lambda v: (0, 0)),
            pl.BlockSpec((1, D), lambda v: (0, 0)),
            pl.BlockSpec((D, V_TILE), lambda v: (0, v)),
            pl.BlockSpec((1, V_TILE), lambda v: (0, v)),
        ],
        out_specs=[
            pl.BlockSpec((B, D), lambda v: (0, 0)),
            pl.BlockSpec((1, B), lambda v: (0, 0)),
        ],
        out_shape=[
            jax.ShapeDtypeStruct((B, D), jnp.float32),
            jax.ShapeDtypeStruct((1, B), jnp.float32),
        ],
        scratch_shapes=[pltpu.VMEM((1, B), jnp.float32)],
    )(s, xT, emb0, WT, ebp)


def _tc_out(m, WT, b, lse):
    def kern(m_ref, w_ref, b_ref, l_ref, o_ref):
        bcol = jnp.transpose(b_ref[...], (1, 0))
        logits = (
            lax.dot_general(
                w_ref[...],
                m_ref[...],
                (((0,), (1,)), ((), ())),
                preferred_element_type=jnp.float32,
            )
            + bcol
        )
        o_ref[...] = logits - l_ref[...]

    return pl.pallas_call(
        kern,
        grid=(NV,),
        in_specs=[
            pl.BlockSpec((B, D), lambda v: (0, 0)),
            pl.BlockSpec((D, V_TILE), lambda v: (0, v)),
            pl.BlockSpec((1, V_TILE), lambda v: (0, v)),
            pl.BlockSpec((1, B), lambda v: (0, 0)),
        ],
        out_specs=pl.BlockSpec((V_TILE, B), lambda v: (v, 0)),
        out_shape=jax.ShapeDtypeStruct((V, B), jnp.float32),
    )(m, WT, b, lse)


def kernel(x, emb, W, b):
    x32 = x.astype(jnp.int32)
    xT = x32.T
    embT = emb.T
    WT = W.T
    b2 = b[None, :]
    embp, ebp = _tc_prep_table(embT, b2)
    s = _sc_embed_sum(xT, embp)
    m, lse = _tc_stats(s, xT, emb[0:1, :], WT, ebp)
    outT = _tc_out(m, WT, b2, lse)
    return outT.T[:, None, :]


# speedup vs baseline: 1.0083x; 1.0083x over previous
import functools

import jax
import jax.numpy as jnp
from jax import lax
from jax.experimental import pallas as pl
from jax.experimental.pallas import tpu as pltpu
from jax.experimental.pallas import tpu_sc as plsc

B, CTX, D, V = 1024, 20, 64, 100000

NC, NS, L = 2, 16, 16
NW = NC * NS
BPW = B // NW
RPW = BPW * CTX
DP = 128

V_TILE = 4096
NV = (V + V_TILE - 1) // V_TILE
LAST_VALID = V - (NV - 1) * V_TILE


VP = NV * V_TILE


def _tc_prep_table(embT, b2):
    def kern(e_ref, b_ref, o_ref, eb_ref):
        o_ref[...] = jnp.concatenate(
            [
                jnp.transpose(e_ref[...], (1, 0)),
                jnp.zeros((V_TILE, DP - D), jnp.float32),
            ],
            axis=1,
        )
        i = pl.program_id(0)
        col = lax.broadcasted_iota(jnp.int32, (1, V_TILE), 1)
        eb_ref[...] = jnp.where(
            col + i * V_TILE < V, jnp.exp(b_ref[...]), 0.0
        )

    return pl.pallas_call(
        kern,
        grid=(NV,),
        in_specs=[
            pl.BlockSpec((D, V_TILE), lambda i: (0, i)),
            pl.BlockSpec((1, V_TILE), lambda i: (0, i)),
        ],
        out_specs=[
            pl.BlockSpec((V_TILE, DP), lambda i: (i, 0)),
            pl.BlockSpec((1, V_TILE), lambda i: (0, i)),
        ],
        out_shape=[
            jax.ShapeDtypeStruct((V, DP), jnp.float32),
            jax.ShapeDtypeStruct((1, VP), jnp.float32),
        ],
    )(embT, b2)


def _sc_embed_sum(xT, embp):
    mesh = plsc.VectorSubcoreMesh(core_axis_name="c", subcore_axis_name="s")

    @functools.partial(
        pl.kernel,
        mesh=mesh,
        out_type=jax.ShapeDtypeStruct((B, D), jnp.float32),
        scratch_types=[
            pltpu.VMEM((CTX, B), jnp.int32),
            pltpu.VMEM((RPW, DP), jnp.float32),
            pltpu.VMEM((BPW, D), jnp.float32),
            pltpu.SemaphoreType.DMA,
        ],
    )
    def k(x_hbm, emb_hbm, out_hbm, idx_v, rows_v, acc_v, sem):
        wid = lax.axis_index("s") * NC + lax.axis_index("c")
        base = wid * BPW
        pltpu.sync_copy(x_hbm, idx_v)
        copies = [
            pltpu.async_copy(
                emb_hbm.at[idx_v.at[j, pl.ds(base, BPW)]],
                rows_v.at[pl.ds(j * BPW, BPW)],
                sem,
            )
            for j in range(CTX)
        ]
        for cp in copies:
            cp.wait()

        def body(bi, carry):
            for g in range(D // L):
                acc = rows_v[bi, pl.ds(g * L, L)]
                for j in range(1, CTX):
                    acc = acc + rows_v[j * BPW + bi, pl.ds(g * L, L)]
                acc_v[bi, pl.ds(g * L, L)] = acc
            return carry

        lax.fori_loop(0, BPW, body, 0)
        pltpu.sync_copy(acc_v, out_hbm.at[pl.ds(base, BPW)])

    return k(xT, embp)


def _tc_softmax(s, xT, emb0, WT, ebp, b2):
    def kern(s_ref, x_ref, e0_ref, w_ref, eb_ref, b_ref, o_ref,
             m_s, lrun, lse_s):
        p = pl.program_id(0)
        v = pl.program_id(1)

        @pl.when((p == 0) & (v == 0))
        def _():
            n0row = jnp.sum(
                jnp.where(x_ref[...] == 0, 1.0, 0.0), axis=0, keepdims=True
            )
            n0 = jnp.transpose(n0row, (1, 0))
            m_s[...] = (s_ref[...] - n0 * e0_ref[...]) * (1.0 / CTX)
            lrun[...] = jnp.zeros((1, B), jnp.float32)

        @pl.when(p == 0)
        def _():
            ex = jnp.exp(
                lax.dot_general(
                    w_ref[...],
                    m_s[...],
                    (((0,), (1,)), ((), ())),
                    preferred_element_type=jnp.float32,
                )
            )
            lrun[...] = lrun[...] + lax.dot_general(
                eb_ref[...],
                ex,
                (((1,), (0,)), ((), ())),
                preferred_element_type=jnp.float32,
            )

            @pl.when(v == NV - 1)
            def _():
                lse_s[...] = jnp.log(lrun[...])

        @pl.when(p == 1)
        def _():
            bcol = jnp.transpose(b_ref[...], (1, 0))
            logits = (
                lax.dot_general(
                    w_ref[...],
                    m_s[...],
                    (((0,), (1,)), ((), ())),
                    preferred_element_type=jnp.float32,
                )
                + bcol
            )
            o_ref[...] = logits - lse_s[...]

    return pl.pallas_call(
        kern,
        grid=(2, NV),
        in_specs=[
            pl.BlockSpec((B, D), lambda p, v: (0, 0)),
            pl.BlockSpec((CTX, B), lambda p, v: (0, 0)),
            pl.BlockSpec((1, D), lambda p, v: (0, 0)),
            pl.BlockSpec((D, V_TILE), lambda p, v: (0, v)),
            pl.BlockSpec((1, V_TILE), lambda p, v: (0, v)),
            pl.BlockSpec((1, V_TILE), lambda p, v: (0, v)),
        ],
        out_specs=pl.BlockSpec((V_TILE, B), lambda p, v: (p * v, 0)),
        out_shape=jax.ShapeDtypeStruct((V, B), jnp.float32),
        scratch_shapes=[
            pltpu.VMEM((B, D), jnp.float32),
            pltpu.VMEM((1, B), jnp.float32),
            pltpu.VMEM((1, B), jnp.float32),
        ],
    )(s, xT, emb0, WT, ebp, b2)


def kernel(x, emb, W, b):
    x32 = x.astype(jnp.int32)
    xT = x32.T
    embT = emb.T
    WT = W.T
    b2 = b[None, :]
    embp, ebp = _tc_prep_table(embT, b2)
    s = _sc_embed_sum(xT, embp)
    outT = _tc_softmax(s, xT, emb[0:1, :], WT, ebp, b2)
    return outT.T[:, None, :]


# speedup vs baseline: 1.0229x; 1.0145x over previous
import functools

import jax
import jax.numpy as jnp
from jax import lax
from jax.experimental import pallas as pl
from jax.experimental.pallas import tpu as pltpu
from jax.experimental.pallas import tpu_sc as plsc

B, CTX, D, V = 1024, 20, 64, 100000

NC, NS, L = 2, 16, 16
NW = NC * NS
BPW = B // NW
RPW = BPW * CTX
DP = 128

V_TILE = 4096
NV = (V + V_TILE - 1) // V_TILE
LAST_VALID = V - (NV - 1) * V_TILE


VP = NV * V_TILE


def _tc_prep_table(embT, b2):
    def kern(e_ref, b_ref, o_ref, eb_ref):
        o_ref[...] = jnp.concatenate(
            [
                jnp.transpose(e_ref[...], (1, 0)),
                jnp.zeros((V_TILE, DP - D), jnp.float32),
            ],
            axis=1,
        )
        i = pl.program_id(0)
        col = lax.broadcasted_iota(jnp.int32, (1, V_TILE), 1)
        eb_ref[...] = jnp.where(
            col + i * V_TILE < V, jnp.exp(b_ref[...]), 0.0
        )

    return pl.pallas_call(
        kern,
        grid=(NV,),
        in_specs=[
            pl.BlockSpec((D, V_TILE), lambda i: (0, i)),
            pl.BlockSpec((1, V_TILE), lambda i: (0, i)),
        ],
        out_specs=[
            pl.BlockSpec((V_TILE, DP), lambda i: (i, 0)),
            pl.BlockSpec((1, V_TILE), lambda i: (0, i)),
        ],
        out_shape=[
            jax.ShapeDtypeStruct((V, DP), jnp.float32),
            jax.ShapeDtypeStruct((1, VP), jnp.float32),
        ],
    )(embT, b2)


def _sc_embed_sum(xT, embp):
    mesh = plsc.VectorSubcoreMesh(core_axis_name="c", subcore_axis_name="s")

    @functools.partial(
        pl.kernel,
        mesh=mesh,
        out_type=jax.ShapeDtypeStruct((B, D), jnp.float32),
        scratch_types=[
            pltpu.VMEM((CTX, 128), jnp.int32),
            pltpu.VMEM((RPW, DP), jnp.float32),
            pltpu.VMEM((BPW, D), jnp.float32),
            pltpu.SemaphoreType.DMA,
        ],
    )
    def k(x_hbm, emb_hbm, out_hbm, idx_v, rows_v, acc_v, sem):
        wid = lax.axis_index("s") * NC + lax.axis_index("c")
        base = wid * BPW
        pltpu.sync_copy(x_hbm.at[:, pl.ds((wid // 4) * 128, 128)], idx_v)
        off = (wid % 4) * BPW
        copies = [
            pltpu.async_copy(
                emb_hbm.at[idx_v.at[j, pl.ds(off, BPW)]],
                rows_v.at[pl.ds(j * BPW, BPW)],
                sem,
            )
            for j in range(CTX)
        ]
        for cp in copies:
            cp.wait()

        def body(bi, carry):
            for g in range(D // L):
                acc = rows_v[bi, pl.ds(g * L, L)]
                for j in range(1, CTX):
                    acc = acc + rows_v[j * BPW + bi, pl.ds(g * L, L)]
                acc_v[bi, pl.ds(g * L, L)] = acc
            return carry

        lax.fori_loop(0, BPW, body, 0)
        pltpu.sync_copy(acc_v, out_hbm.at[pl.ds(base, BPW)])

    return k(xT, embp)


def _tc_softmax(s, xT, emb0, WT, ebp, b2):
    LOG2E = 1.4426950408889634

    def kern(s_ref, x_ref, e0_ref, w_ref, eb_ref, b_ref, o_ref,
             m_s, m2_s, lrun, lse_s):
        p = pl.program_id(0)
        v = pl.program_id(1)

        @pl.when((p == 0) & (v == 0))
        def _():
            n0row = jnp.sum(
                jnp.where(x_ref[...] == 0, 1.0, 0.0), axis=0, keepdims=True
            )
            n0 = jnp.transpose(n0row, (1, 0))
            m = (s_ref[...] - n0 * e0_ref[...]) * (1.0 / CTX)
            m_s[...] = m
            m2_s[...] = m * LOG2E
            lrun[...] = jnp.zeros((1, B), jnp.float32)

        @pl.when(p == 0)
        def _():
            ex = jnp.exp2(
                lax.dot_general(
                    w_ref[...],
                    m2_s[...],
                    (((0,), (1,)), ((), ())),
                    preferred_element_type=jnp.float32,
                )
            )
            lrun[...] = lrun[...] + lax.dot_general(
                eb_ref[...],
                ex,
                (((1,), (0,)), ((), ())),
                preferred_element_type=jnp.float32,
            )

            @pl.when(v == NV - 1)
            def _():
                lse_s[...] = jnp.log(lrun[...])

        @pl.when(p == 1)
        def _():
            bcol = jnp.transpose(b_ref[...], (1, 0))
            logits = (
                lax.dot_general(
                    w_ref[...],
                    m_s[...],
                    (((0,), (1,)), ((), ())),
                    preferred_element_type=jnp.float32,
                )
                + bcol
            )
            o_ref[...] = logits - lse_s[...]

    return pl.pallas_call(
        kern,
        grid=(2, NV),
        in_specs=[
            pl.BlockSpec((B, D), lambda p, v: (0, 0)),
            pl.BlockSpec((CTX, B), lambda p, v: (0, 0)),
            pl.BlockSpec((1, D), lambda p, v: (0, 0)),
            pl.BlockSpec((D, V_TILE), lambda p, v: (0, v)),
            pl.BlockSpec((1, V_TILE), lambda p, v: (0, v)),
            pl.BlockSpec((1, V_TILE), lambda p, v: (0, v)),
        ],
        out_specs=pl.BlockSpec((V_TILE, B), lambda p, v: (p * v, 0)),
        out_shape=jax.ShapeDtypeStruct((V, B), jnp.float32),
        scratch_shapes=[
            pltpu.VMEM((B, D), jnp.float32),
            pltpu.VMEM((B, D), jnp.float32),
            pltpu.VMEM((1, B), jnp.float32),
            pltpu.VMEM((1, B), jnp.float32),
        ],
    )(s, xT, emb0, WT, ebp, b2)


def kernel(x, emb, W, b):
    x32 = x.astype(jnp.int32)
    xT = x32.T
    embT = emb.T
    WT = W.T
    b2 = b[None, :]
    embp, ebp = _tc_prep_table(embT, b2)
    s = _sc_embed_sum(xT, embp)
    outT = _tc_softmax(s, xT, emb[0:1, :], WT, ebp, b2)
    return outT.T[:, None, :]
